# Initial kernel scaffold; baseline (speedup 1.0000x reference)
import jax
import jax.numpy as jnp
from jax.experimental import pallas as pl


def kernel(x, edge_index, W1, b1, W2, b2, W3, b3):
    raise NotImplementedError("write your pallas kernel here")


# speedup vs baseline: 11.4490x; 11.4490x over previous
import functools

import jax
import jax.numpy as jnp
from jax import lax
from jax.experimental import pallas as pl
from jax.experimental.pallas import tpu as pltpu
from jax.experimental.pallas import tpu_sc as plsc

N = 10000
D = 128
E = 320000
NC = 2
NS = 16
NW = NC * NS
CHUNK = 128
NCH = -(-E // (NW * CHUNK))
EPT = NCH * CHUNK
EPAD = NW * EPT
NPAD = 10240
TRASH = N
ROWS_PT = NPAD // NS
DEGW = 16

_MESH = dict(core_axis_name="c", subcore_axis_name="s")


def _deg_sc(dst_t):
    @functools.partial(
        pl.kernel,
        out_type=jax.ShapeDtypeStruct((NC, NPAD, DEGW), jnp.float32),
        mesh=plsc.VectorSubcoreMesh(**_MESH),
        scratch_types=[
            pltpu.VMEM((NCH, CHUNK), jnp.int32),
            pltpu.VMEM((CHUNK, DEGW), jnp.float32),
            pltpu.VMEM((CHUNK, DEGW), jnp.float32),
            pltpu.VMEM_SHARED((NPAD, DEGW), jnp.float32),
        ],
    )
    def k(dst_hbm, out_hbm, dst_v, ones_v, zeros_v, dacc):
        c = lax.axis_index("c")
        s = lax.axis_index("s")
        wid = c * NS + s
        pltpu.sync_copy(dst_hbm.at[wid], dst_v)
        one16 = jnp.ones((16,), jnp.float32)
        zero16 = jnp.zeros((16,), jnp.float32)

        def fill(r, _):
            ones_v[r, :] = one16
            zeros_v[r, :] = zero16
            return 0

        lax.fori_loop(0, CHUNK, fill, 0)

        base = s * ROWS_PT
        for kk in range(ROWS_PT // CHUNK):
            pltpu.sync_copy(zeros_v, dacc.at[pl.ds(base + kk * CHUNK, CHUNK)])
        plsc.subcore_barrier()

        def add1(j, _):
            pltpu.sync_copy(ones_v, dacc.at[dst_v.at[j]], add=True)
            return 0

        lax.fori_loop(0, NCH, add1, 0)
        plsc.subcore_barrier()
        pltpu.sync_copy(dacc.at[pl.ds(base, ROWS_PT)],
                        out_hbm.at[c, pl.ds(base, ROWS_PT)])

    return k(dst_t)


def _spmm_sc(hp, src_t, dst_t):
    @functools.partial(
        pl.kernel,
        out_type=jax.ShapeDtypeStruct((NC, NPAD, D), jnp.float32),
        mesh=plsc.VectorSubcoreMesh(**_MESH),
        scratch_types=[
            pltpu.VMEM((NCH, CHUNK), jnp.int32),
            pltpu.VMEM((NCH, CHUNK), jnp.int32),
            pltpu.VMEM((CHUNK, D), jnp.float32),
            pltpu.VMEM_SHARED((NPAD, D), jnp.float32),
            pltpu.SemaphoreType.DMA,
        ],
    )
    def k(hp_hbm, src_hbm, dst_hbm, out_hbm, src_v, dst_v, rows_v, acc, sem):
        c = lax.axis_index("c")
        s = lax.axis_index("s")
        wid = c * NS + s
        pltpu.sync_copy(src_hbm.at[wid], src_v)
        pltpu.sync_copy(dst_hbm.at[wid], dst_v)
        base = s * ROWS_PT

        @pl.when(c == 0)
        def _():
            pltpu.sync_copy(hp_hbm.at[pl.ds(base, ROWS_PT)],
                            acc.at[pl.ds(base, ROWS_PT)])

        @pl.when(c != 0)
        def _():
            zero16 = jnp.zeros((16,), jnp.float32)

            def zr(r, _):
                for kk in range(D // 16):
                    rows_v[r, pl.ds(kk * 16, 16)] = zero16
                return 0

            lax.fori_loop(0, CHUNK, zr, 0)
            for kk in range(ROWS_PT // CHUNK):
                pltpu.sync_copy(rows_v,
                                acc.at[pl.ds(base + kk * CHUNK, CHUNK)])

        plsc.subcore_barrier()

        def edge_chunk(j, _):
            pltpu.async_copy(hp_hbm.at[src_v.at[j]], rows_v, sem).wait()
            pltpu.sync_copy(rows_v, acc.at[dst_v.at[j]], add=True)
            return 0

        lax.fori_loop(0, NCH, edge_chunk, 0)
        plsc.subcore_barrier()
        pltpu.sync_copy(acc.at[pl.ds(base, ROWS_PT)],
                        out_hbm.at[c, pl.ds(base, ROWS_PT)])

    return k(hp, src_t, dst_t)


BR = 512


def _tc_first(x, W1, degp):
    def body(x_ref, w_ref, dp_ref, hp_ref, dv_ref):
        dp = dp_ref[...]
        deg = dp[0, :, 0:1] + dp[1, :, 0:1] + 1.0
        dinv = lax.rsqrt(deg)
        dv = jnp.broadcast_to(dinv, (BR, D))
        dv_ref[...] = dv
        hp_ref[...] = dv * jnp.dot(x_ref[...], w_ref[...],
                                   preferred_element_type=jnp.float32)

    return pl.pallas_call(
        body,
        grid=(NPAD // BR,),
        in_specs=[
            pl.BlockSpec((BR, D), lambda i: (i, 0)),
            pl.BlockSpec((D, D), lambda i: (0, 0)),
            pl.BlockSpec((NC, BR, DEGW), lambda i: (0, i, 0)),
        ],
        out_specs=[pl.BlockSpec((BR, D), lambda i: (i, 0)),
                   pl.BlockSpec((BR, D), lambda i: (i, 0))],
        out_shape=[jax.ShapeDtypeStruct((NPAD, D), jnp.float32),
                   jax.ShapeDtypeStruct((NPAD, D), jnp.float32)],
    )(x, W1, degp)


def _tc_mid(p, dinv2, b_row, W):
    def body(p_ref, dv_ref, b_ref, w_ref, hp_ref):
        pp = p_ref[...]
        dv = dv_ref[...]
        h = jax.nn.relu(dv * (pp[0] + pp[1]) + b_ref[...])
        hp_ref[...] = dv * jnp.dot(h, w_ref[...],
                                   preferred_element_type=jnp.float32)

    return pl.pallas_call(
        body,
        grid=(NPAD // BR,),
        in_specs=[
            pl.BlockSpec((NC, BR, D), lambda i: (0, i, 0)),
            pl.BlockSpec((BR, D), lambda i: (i, 0)),
            pl.BlockSpec((1, D), lambda i: (0, 0)),
            pl.BlockSpec((D, D), lambda i: (0, 0)),
        ],
        out_specs=pl.BlockSpec((BR, D), lambda i: (i, 0)),
        out_shape=jax.ShapeDtypeStruct((NPAD, D), jnp.float32),
    )(p, dinv2, b_row, W)


def _tc_last(p, dinv2, b_row):
    def body(p_ref, dv_ref, b_ref, o_ref):
        pp = p_ref[...]
        o_ref[...] = dv_ref[...] * (pp[0] + pp[1]) + b_ref[...]

    return pl.pallas_call(
        body,
        grid=(NPAD // BR,),
        in_specs=[
            pl.BlockSpec((NC, BR, D), lambda i: (0, i, 0)),
            pl.BlockSpec((BR, D), lambda i: (i, 0)),
            pl.BlockSpec((1, D), lambda i: (0, 0)),
        ],
        out_specs=pl.BlockSpec((BR, D), lambda i: (i, 0)),
        out_shape=jax.ShapeDtypeStruct((NPAD, D), jnp.float32),
    )(p, dinv2, b_row)


def kernel(x, edge_index, W1, b1, W2, b2, W3, b3):
    pad = EPAD - E
    src_t = jnp.concatenate(
        [edge_index[0], jnp.zeros((pad,), jnp.int32)]).reshape(NW, NCH, CHUNK)
    dst_t = jnp.concatenate(
        [edge_index[1], jnp.full((pad,), TRASH, jnp.int32)]).reshape(NW, NCH, CHUNK)

    x_p = jnp.pad(x, ((0, NPAD - N), (0, 0)))
    degp = _deg_sc(dst_t)
    hp, dinv2 = _tc_first(x_p, W1, degp)
    p = _spmm_sc(hp, src_t, dst_t)
    hp = _tc_mid(p, dinv2, b1.reshape(1, D), W2)
    p = _spmm_sc(hp, src_t, dst_t)
    hp = _tc_mid(p, dinv2, b2.reshape(1, D), W3)
    p = _spmm_sc(hp, src_t, dst_t)
    return _tc_last(p, dinv2, b3.reshape(1, D))[:N]
